# baseline (speedup 1.0000x reference)
import jax
import jax.numpy as jnp
from jax import lax
from jax.experimental import pallas as pl
from jax.experimental.pallas import tpu as pltpu
from jax.experimental.pallas import tpu_sc as plsc

EMB = 64
T = 200
B = 4096
NC, NS, L = 2, 16, 16
NW = NC * NS
BB = B // NW
NBUF = 4
NOB = 2


def _body(x_hbm, tpair_hbm, pos_hbm, out_hbm,
          xblk, posv, idx2v,
          buf0, buf1, buf2, buf3, tb0, tb1,
          gs0, gs1, gs2, gs3, os0, os1):
    bufs = (buf0, buf1, buf2, buf3)
    tbufs = (tb0, tb1)
    gsems = (gs0, gs1, gs2, gs3)
    osems = (os0, os1)

    w = lax.axis_index("s") * NC + lax.axis_index("c")

    pltpu.sync_copy(x_hbm.at[w], xblk)
    pltpu.sync_copy(pos_hbm.at[pl.ds(0, T)], posv)

    iota = lax.iota(jnp.int32, L)
    rivs = [iota + bb * L for bb in range(BB // L)]

    def idx2_prep(t, b):
        for bb in range(BB // L):
            sl = pl.ds(bb * L, L)
            idx2v[b, sl] = lax.shift_right_logical(xblk[t, sl], 1)

    def gather_start(b):
        pltpu.async_copy(tpair_hbm.at[idx2v.at[b]], bufs[b], gsems[b])

    def gather_wait(b):
        pltpu.make_async_copy(tpair_hbm.at[idx2v.at[b]], bufs[b],
                              gsems[b]).wait()

    def out_start(t, tb):
        pltpu.async_copy(tbufs[tb], out_hbm.at[t, :, w], osems[tb])

    def out_wait(t, tb):
        pltpu.make_async_copy(tbufs[tb], out_hbm.at[t, :, w],
                              osems[tb]).wait()

    for b in range(NBUF):
        idx2_prep(b, b)
        gather_start(b)

    def group(g, _):
        for b in range(NBUF):
            t = g * NBUF + b
            tb = b % NOB
            gather_wait(b)

            cbs = []
            for bb in range(BB // L):
                xv = xblk[t, pl.ds(bb * L, L)]
                cbs.append(lax.shift_left(jnp.bitwise_and(xv, 1), 6))

            def make_col(q, prq, _b=b, _tb=tb, _cbs=cbs):
                def col(c2, _c):
                    pc = prq[jnp.full((L,), c2, jnp.int32)]
                    c = q * L + c2
                    cg = lax.div(c, 8)
                    cr = lax.rem(c, 8)
                    for bb in range(BB // L):
                        v = plsc.load_gather(bufs[_b],
                                             [rivs[bb], _cbs[bb] + c])
                        tbufs[_tb][cg, cr, pl.ds(bb * L, L)] = v + pc
                    return _c
                return col

            if b < NOB:
                @pl.when(g > 0)
                def _():
                    out_wait(t, tb)
            else:
                out_wait(t, tb)
            for q in range(EMB // L):
                prq = posv[t, pl.ds(q * L, L)]
                lax.fori_loop(0, L, make_col(q, prq), 0, unroll=2)
            out_start(t, tb)

            @pl.when(g < (T // NBUF) - 1)
            def _():
                idx2_prep(t + NBUF, b)
                gather_start(b)
        return 0

    lax.fori_loop(0, T // NBUF, group, 0)

    for b in range(NOB):
        out_wait(T - NOB + b, b % NOB)


@jax.jit
def kernel(x, table, pos_emb):
    xw = x.T.reshape(T, NW, BB).transpose(1, 0, 2)
    tpair = table.reshape(table.shape[0] // 2, 2 * EMB)
    post = pos_emb

    kfn = pl.kernel(
        _body,
        out_type=jax.ShapeDtypeStruct((T, 8, NW, 8, BB), jnp.float32),
        compiler_params=pltpu.CompilerParams(use_tc_tiling_on_sc=False,
                                             needs_layout_passes=False),
        mesh=plsc.VectorSubcoreMesh(
            core_axis_name="c", subcore_axis_name="s",
            num_cores=NC, num_subcores=NS),
        scratch_types=[
            pltpu.VMEM((T, BB), jnp.int32),
            pltpu.VMEM((T, EMB), jnp.float32),
            pltpu.VMEM((NBUF, BB), jnp.int32),
            pltpu.VMEM((BB, 2 * EMB), jnp.float32),
            pltpu.VMEM((BB, 2 * EMB), jnp.float32),
            pltpu.VMEM((BB, 2 * EMB), jnp.float32),
            pltpu.VMEM((BB, 2 * EMB), jnp.float32),
            pltpu.VMEM((8, 8, BB), jnp.float32),
            pltpu.VMEM((8, 8, BB), jnp.float32),
        ] + [pltpu.SemaphoreType.DMA] * (NBUF + NOB),
    )
    out = kfn(xw, tpair, post)
    return out.transpose(2, 4, 0, 1, 3).reshape(B, T, EMB)


# speedup vs baseline: 1.4831x; 1.4831x over previous
import jax
import jax.numpy as jnp
from jax import lax
from jax.experimental import pallas as pl
from jax.experimental.pallas import tpu as pltpu
from jax.experimental.pallas import tpu_sc as plsc

EMB = 64
T = 200
B = 4096
NC, NS, L = 2, 16, 16
NW = NC * NS
BB = B // NW
NBUF = 4
NOB = 2


def _body(x_hbm, tpair_hbm, pos_hbm, out_hbm,
          xblk, posv, idx2v,
          buf0, buf1, buf2, buf3, tb0, tb1,
          gs0, gs1, gs2, gs3, os0, os1):
    bufs = (buf0, buf1, buf2, buf3)
    tbufs = (tb0, tb1)
    gsems = (gs0, gs1, gs2, gs3)
    osems = (os0, os1)

    w = lax.axis_index("s") * NC + lax.axis_index("c")

    pltpu.sync_copy(x_hbm.at[w], xblk)
    pltpu.sync_copy(pos_hbm.at[pl.ds(0, T)], posv)

    iota = lax.iota(jnp.int32, L)
    rivs = [iota + bb * L for bb in range(BB // L)]

    def idx2_prep(t, b):
        for bb in range(BB // L):
            sl = pl.ds(bb * L, L)
            idx2v[b, sl] = lax.shift_right_logical(xblk[t, sl], 1)

    def gather_start(b):
        pltpu.async_copy(tpair_hbm.at[idx2v.at[b]], bufs[b], gsems[b])

    def gather_wait(b):
        pltpu.make_async_copy(tpair_hbm.at[idx2v.at[b]], bufs[b],
                              gsems[b]).wait()

    def out_start(t, tb):
        pltpu.async_copy(tbufs[tb], out_hbm.at[t, :, w], osems[tb])

    def out_wait(t, tb):
        pltpu.make_async_copy(tbufs[tb], out_hbm.at[t, :, w],
                              osems[tb]).wait()

    for b in range(NBUF):
        idx2_prep(b, b)
        gather_start(b)

    def group(g, _):
        for b in range(NBUF):
            t = g * NBUF + b
            tb = b % NOB
            gather_wait(b)

            cbs = []
            for bb in range(BB // L):
                xv = xblk[t, pl.ds(bb * L, L)]
                cbs.append(lax.shift_left(jnp.bitwise_and(xv, 1), 6))

            if b < NOB:
                @pl.when(g > 0)
                def _():
                    out_wait(t, tb)
            else:
                out_wait(t, tb)
            for q in range(EMB // L):
                prq = posv[t, pl.ds(q * L, L)]
                cqs = [cbs[bb] + (q * L) for bb in range(BB // L)]

                @plsc.parallel_loop(0, L, 1, unroll=4)
                def _col(c2, _q=q, _prq=prq, _cqs=cqs, _b=b, _tb=tb):
                    pc = _prq[jnp.full((L,), c2, jnp.int32)]
                    cg = _q * 2 + lax.div(c2, 8)
                    cr = lax.rem(c2, 8)
                    for bb in range(BB // L):
                        v = plsc.load_gather(bufs[_b],
                                             [rivs[bb], _cqs[bb] + c2])
                        tbufs[_tb][cg, cr, pl.ds(bb * L, L)] = v + pc
            out_start(t, tb)

            @pl.when(g < (T // NBUF) - 1)
            def _():
                idx2_prep(t + NBUF, b)
                gather_start(b)
        return 0

    lax.fori_loop(0, T // NBUF, group, 0)

    for b in range(NOB):
        out_wait(T - NOB + b, b % NOB)


@jax.jit
def kernel(x, table, pos_emb):
    xw = x.T.reshape(T, NW, BB).transpose(1, 0, 2)
    tpair = table.reshape(table.shape[0] // 2, 2 * EMB)
    post = pos_emb

    kfn = pl.kernel(
        _body,
        out_type=jax.ShapeDtypeStruct((T, 8, NW, 8, BB), jnp.float32),
        compiler_params=pltpu.CompilerParams(use_tc_tiling_on_sc=False,
                                             needs_layout_passes=False),
        mesh=plsc.VectorSubcoreMesh(
            core_axis_name="c", subcore_axis_name="s",
            num_cores=NC, num_subcores=NS),
        scratch_types=[
            pltpu.VMEM((T, BB), jnp.int32),
            pltpu.VMEM((T, EMB), jnp.float32),
            pltpu.VMEM((NBUF, BB), jnp.int32),
            pltpu.VMEM((BB, 2 * EMB), jnp.float32),
            pltpu.VMEM((BB, 2 * EMB), jnp.float32),
            pltpu.VMEM((BB, 2 * EMB), jnp.float32),
            pltpu.VMEM((BB, 2 * EMB), jnp.float32),
            pltpu.VMEM((8, 8, BB), jnp.float32),
            pltpu.VMEM((8, 8, BB), jnp.float32),
        ] + [pltpu.SemaphoreType.DMA] * (NBUF + NOB),
    )
    out = kfn(xw, tpair, post)
    return out.transpose(2, 4, 0, 1, 3).reshape(B, T, EMB)
